# baseline (speedup 1.0000x reference)
import functools

import jax
import jax.numpy as jnp
from jax import lax
from jax.experimental import pallas as pl
from jax.experimental.pallas import tpu as pltpu
from jax.experimental.pallas import tpu_sc as plsc

R, C = 16384, 1024
RNG_LEN = 1024
NC, NS = 2, 16
NW = NC * NS
ROWS_W = R // NW
BR = 8
NCHUNK = ROWS_W // BR
NBUF = 4
LANES = 16
VECS = BR * C // LANES


def _body(bin_hbm, cyc_hbm, rng_hbm, out_hbm, rng_v, bin_v, cyc_v, out_v,
          sin_b, sin_c, sout):
    wid = lax.axis_index("s") * NC + lax.axis_index("c")
    base = wid * ROWS_W

    pltpu.sync_copy(rng_hbm, rng_v)

    def start_in(g, b):
        r0 = base + g * BR
        pltpu.async_copy(bin_hbm.at[pl.ds(r0, BR)], bin_v.at[b], sin_b[b])
        pltpu.async_copy(cyc_hbm.at[pl.ds(r0, BR)], cyc_v.at[b], sin_c[b])

    def wait_in(g, b):
        r0 = base + g * BR
        pltpu.make_async_copy(bin_hbm.at[pl.ds(r0, BR)], bin_v.at[b],
                              sin_b[b]).wait()
        pltpu.make_async_copy(cyc_hbm.at[pl.ds(r0, BR)], cyc_v.at[b],
                              sin_c[b]).wait()

    def wait_out(g, b):
        r0 = base + g * BR
        pltpu.make_async_copy(out_v.at[b], out_hbm.at[pl.ds(r0, BR)],
                              sout[b]).wait()

    start_in(0, 0)

    def super_body(gg, _):
        g0 = gg * NBUF
        for b in range(NBUF):
            g = g0 + b
            nb = (b + 1) % NBUF

            @pl.when(g + 1 < NCHUNK)
            def _():
                start_in(g + 1, nb)

            @pl.when(g >= NBUF)
            def _():
                wait_out(g - NBUF, b)

            wait_in(g, b)

            def vec_body(v):
                r = v // (C // LANES)
                s = (v % (C // LANES)) * LANES
                cyc = cyc_v[b, r, pl.ds(s, LANES)]
                idx = lax.bitwise_and(cyc, RNG_LEN - 1)
                g16 = plsc.load_gather(rng_v, [idx])
                b16 = bin_v[b, r, pl.ds(s, LANES)]
                out_v[b, r, pl.ds(s, LANES)] = jnp.where(
                    b16 > g16, 1.0, 0.0).astype(jnp.float32)

            plsc.parallel_loop(0, VECS, 1, unroll=16, carry=None)(vec_body)

            r0 = base + g * BR
            pltpu.async_copy(out_v.at[b], out_hbm.at[pl.ds(r0, BR)], sout[b])
        return 0

    lax.fori_loop(0, NCHUNK // NBUF, super_body, 0)
    for b in range(NBUF):
        wait_out(NCHUNK - NBUF + b, b)


@jax.jit
def kernel(binary, rng, cycle):
    mesh = plsc.VectorSubcoreMesh(
        core_axis_name="c", subcore_axis_name="s", num_cores=NC,
        num_subcores=NS)
    run = functools.partial(
        pl.kernel,
        out_type=jax.ShapeDtypeStruct((R, C), jnp.float32),
        mesh=mesh,
        scratch_types=[
            pltpu.VMEM((RNG_LEN,), jnp.float32),
            pltpu.VMEM((NBUF, BR, C), jnp.float32),
            pltpu.VMEM((NBUF, BR, C), jnp.int32),
            pltpu.VMEM((NBUF, BR, C), jnp.float32),
            [pltpu.SemaphoreType.DMA] * NBUF,
            [pltpu.SemaphoreType.DMA] * NBUF,
            [pltpu.SemaphoreType.DMA] * NBUF,
        ],
        compiler_params=pltpu.CompilerParams(
            needs_layout_passes=False, use_tc_tiling_on_sc=True),
    )(_body)
    return run(binary, cycle, rng)


# speedup vs baseline: 1.0869x; 1.0869x over previous
import functools

import jax
import jax.numpy as jnp
from jax import lax
from jax.experimental import pallas as pl
from jax.experimental.pallas import tpu as pltpu
from jax.experimental.pallas import tpu_sc as plsc

R, C = 16384, 1024
RNG_LEN = 1024
NC, NS = 2, 16
NW = NC * NS
ROWS_W = R // NW
BR = 16
NCHUNK = ROWS_W // BR
NBUF = 2
LANES = 16
VECS = BR * C // LANES


def _body(bin_hbm, cyc_hbm, rng_hbm, out_hbm, rng_v, bin_v, cyc_v, out_v,
          sin_b, sin_c, sout):
    wid = lax.axis_index("s") * NC + lax.axis_index("c")
    base = wid * ROWS_W

    pltpu.sync_copy(rng_hbm, rng_v)

    def start_in(g, b):
        r0 = base + g * BR
        pltpu.async_copy(bin_hbm.at[pl.ds(r0, BR)], bin_v.at[b], sin_b[b])
        pltpu.async_copy(cyc_hbm.at[pl.ds(r0, BR)], cyc_v.at[b], sin_c[b])

    def wait_in(g, b):
        r0 = base + g * BR
        pltpu.make_async_copy(bin_hbm.at[pl.ds(r0, BR)], bin_v.at[b],
                              sin_b[b]).wait()
        pltpu.make_async_copy(cyc_hbm.at[pl.ds(r0, BR)], cyc_v.at[b],
                              sin_c[b]).wait()

    def wait_out(g, b):
        r0 = base + g * BR
        pltpu.make_async_copy(out_v.at[b], out_hbm.at[pl.ds(r0, BR)],
                              sout[b]).wait()

    start_in(0, 0)

    def super_body(gg, _):
        g0 = gg * NBUF
        for b in range(NBUF):
            g = g0 + b
            nb = (b + 1) % NBUF

            @pl.when(g + 1 < NCHUNK)
            def _():
                start_in(g + 1, nb)

            @pl.when(g >= NBUF)
            def _():
                wait_out(g - NBUF, b)

            wait_in(g, b)

            bv, cv, ov = bin_v.at[b], cyc_v.at[b], out_v.at[b]

            def vec_body(v):
                r = lax.shift_right_logical(v, 6)
                s = lax.shift_left(lax.bitwise_and(v, 63), 4)
                cyc = cv[r, pl.ds(s, LANES)]
                idx = lax.bitwise_and(cyc, RNG_LEN - 1)
                g16 = plsc.load_gather(rng_v, [idx])
                b16 = bv[r, pl.ds(s, LANES)]
                ov[r, pl.ds(s, LANES)] = jnp.where(b16 > g16, 1.0, 0.0)

            plsc.parallel_loop(0, VECS, 1, unroll=16, carry=None)(vec_body)

            r0 = base + g * BR
            pltpu.async_copy(out_v.at[b], out_hbm.at[pl.ds(r0, BR)], sout[b])
        return 0

    lax.fori_loop(0, NCHUNK // NBUF, super_body, 0)
    for b in range(NBUF):
        wait_out(NCHUNK - NBUF + b, b)


@jax.jit
def kernel(binary, rng, cycle):
    mesh = plsc.VectorSubcoreMesh(
        core_axis_name="c", subcore_axis_name="s", num_cores=NC,
        num_subcores=NS)
    run = functools.partial(
        pl.kernel,
        out_type=jax.ShapeDtypeStruct((R, C), jnp.float32),
        mesh=mesh,
        scratch_types=[
            pltpu.VMEM((RNG_LEN,), jnp.float32),
            pltpu.VMEM((NBUF, BR, C), jnp.float32),
            pltpu.VMEM((NBUF, BR, C), jnp.int32),
            pltpu.VMEM((NBUF, BR, C), jnp.float32),
            [pltpu.SemaphoreType.DMA] * NBUF,
            [pltpu.SemaphoreType.DMA] * NBUF,
            [pltpu.SemaphoreType.DMA] * NBUF,
        ],
        compiler_params=pltpu.CompilerParams(
            needs_layout_passes=False, use_tc_tiling_on_sc=True),
    )(_body)
    return run(binary, cycle, rng)


# speedup vs baseline: 1.1063x; 1.0178x over previous
import functools

import jax
import jax.numpy as jnp
from jax import lax
from jax.experimental import pallas as pl
from jax.experimental.pallas import tpu as pltpu
from jax.experimental.pallas import tpu_sc as plsc

R, C = 16384, 1024
RNG_LEN = 1024
NC, NS = 2, 16
NW = NC * NS
ROWS_W = R // NW
BR = 16
NCHUNK = ROWS_W // BR
NBUF = 2
LANES = 16
VECS = BR * C // LANES


def _body(bin_hbm, cyc_hbm, rng_hbm, out_hbm, rng_v, bin_v, cyc_v, out_v,
          sin_b, sin_c, sout):
    wid = lax.axis_index("s") * NC + lax.axis_index("c")
    base = wid * ROWS_W

    def start_in(g, b):
        r0 = base + g * BR
        pltpu.async_copy(bin_hbm.at[pl.ds(r0, BR)], bin_v.at[b], sin_b[b])
        pltpu.async_copy(cyc_hbm.at[pl.ds(r0, BR)], cyc_v.at[b], sin_c[b])

    def wait_in(g, b):
        r0 = base + g * BR
        pltpu.make_async_copy(bin_hbm.at[pl.ds(r0, BR)], bin_v.at[b],
                              sin_b[b]).wait()
        pltpu.make_async_copy(cyc_hbm.at[pl.ds(r0, BR)], cyc_v.at[b],
                              sin_c[b]).wait()

    def wait_out(g, b):
        r0 = base + g * BR
        pltpu.make_async_copy(out_v.at[b], out_hbm.at[pl.ds(r0, BR)],
                              sout[b]).wait()

    start_in(0, 0)
    pltpu.sync_copy(rng_hbm, rng_v)

    def super_body(gg, _):
        g0 = gg * NBUF
        for b in range(NBUF):
            g = g0 + b
            nb = (b + 1) % NBUF

            @pl.when(g + 1 < NCHUNK)
            def _():
                start_in(g + 1, nb)

            @pl.when(g >= NBUF)
            def _():
                wait_out(g - NBUF, b)

            wait_in(g, b)

            bv, cv, ov = bin_v.at[b], cyc_v.at[b], out_v.at[b]

            def vec_body(v):
                r = lax.shift_right_logical(v, 6)
                s = lax.shift_left(lax.bitwise_and(v, 63), 4)
                cyc = cv[r, pl.ds(s, LANES)]
                idx = lax.bitwise_and(cyc, RNG_LEN - 1)
                g16 = plsc.load_gather(rng_v, [idx])
                b16 = bv[r, pl.ds(s, LANES)]
                ov[r, pl.ds(s, LANES)] = jnp.where(b16 > g16, 1.0, 0.0)

            plsc.parallel_loop(0, VECS, 1, unroll=16, carry=None)(vec_body)

            r0 = base + g * BR
            pltpu.async_copy(out_v.at[b], out_hbm.at[pl.ds(r0, BR)], sout[b])
        return 0

    lax.fori_loop(0, NCHUNK // NBUF, super_body, 0)
    for b in range(NBUF):
        wait_out(NCHUNK - NBUF + b, b)


@jax.jit
def kernel(binary, rng, cycle):
    mesh = plsc.VectorSubcoreMesh(
        core_axis_name="c", subcore_axis_name="s", num_cores=NC,
        num_subcores=NS)
    run = functools.partial(
        pl.kernel,
        out_type=jax.ShapeDtypeStruct((R, C), jnp.float32),
        mesh=mesh,
        scratch_types=[
            pltpu.VMEM((RNG_LEN,), jnp.float32),
            pltpu.VMEM((NBUF, BR, C), jnp.float32),
            pltpu.VMEM((NBUF, BR, C), jnp.int32),
            pltpu.VMEM((NBUF, BR, C), jnp.float32),
            [pltpu.SemaphoreType.DMA] * NBUF,
            [pltpu.SemaphoreType.DMA] * NBUF,
            [pltpu.SemaphoreType.DMA] * NBUF,
        ],
        compiler_params=pltpu.CompilerParams(
            needs_layout_passes=False, use_tc_tiling_on_sc=True),
    )(_body)
    return run(binary, cycle, rng)
